# Initial kernel scaffold; baseline (speedup 1.0000x reference)
import jax
import jax.numpy as jnp
from jax.experimental import pallas as pl


def kernel(class_ids, api_ids, class_table, api_table):
    raise NotImplementedError("write your pallas kernel here")


# speedup vs baseline: 2.6423x; 2.6423x over previous
import functools
import math

import jax
import jax.numpy as jnp
from jax import lax
from jax.experimental import pallas as pl
from jax.experimental.pallas import tpu as pltpu
from jax.experimental.pallas import tpu_sc as plsc

API_DIM = 32
CLASS_DIM = 32
FINAL_DIM = API_DIM + CLASS_DIM
SCALE = math.sqrt(FINAL_DIM)

NC = 2
NS = 16
NW = NC * NS
CHUNK = 128


def _sc_embed(n_rows, n_chunks_per_w):
    rows_per_w = n_chunks_per_w * CHUNK
    mesh = plsc.VectorSubcoreMesh(core_axis_name="c", subcore_axis_name="s")

    @functools.partial(
        pl.kernel,
        out_type=jax.ShapeDtypeStruct((n_rows, FINAL_DIM), jnp.float32),
        mesh=mesh,
        scratch_types=[
            pltpu.VMEM((n_chunks_per_w, CHUNK), jnp.int32),
            pltpu.VMEM((n_chunks_per_w, CHUNK), jnp.int32),
            pltpu.VMEM((CHUNK, CLASS_DIM), jnp.float32),
            pltpu.VMEM((CHUNK, API_DIM), jnp.float32),
            pltpu.VMEM((CHUNK, FINAL_DIM), jnp.float32),
            pltpu.SemaphoreType.DMA,
        ],
        compiler_params=pltpu.CompilerParams(use_tc_tiling_on_sc=False),
    )
    def k(cls_ids, api_ids, cls_tab, api_tab, out,
          idx_cls, idx_api, cls_v, api_v, out_v, sem):
        wid = lax.axis_index("s") * NC + lax.axis_index("c")
        cbase = wid * n_chunks_per_w
        pltpu.sync_copy(cls_ids.at[pl.ds(cbase, n_chunks_per_w)], idx_cls)
        pltpu.sync_copy(api_ids.at[pl.ds(cbase, n_chunks_per_w)], idx_api)

        def chunk_body(j, carry):
            pltpu.async_copy(cls_tab.at[idx_cls.at[j]], cls_v, sem).wait()
            pltpu.async_copy(api_tab.at[idx_api.at[j]], api_v, sem).wait()

            def row_body(i, c):
                out_v[i, pl.ds(0, 16)] = cls_v[i, pl.ds(0, 16)] * SCALE
                out_v[i, pl.ds(16, 16)] = cls_v[i, pl.ds(16, 16)] * SCALE
                out_v[i, pl.ds(32, 16)] = api_v[i, pl.ds(0, 16)] * SCALE
                out_v[i, pl.ds(48, 16)] = api_v[i, pl.ds(16, 16)] * SCALE
                return c

            lax.fori_loop(0, CHUNK, row_body, 0)
            pltpu.sync_copy(out_v,
                            out.at[pl.ds(wid * rows_per_w + j * CHUNK, CHUNK)])
            return carry

        lax.fori_loop(0, n_chunks_per_w, chunk_body, 0)

    return k


def kernel(class_ids, api_ids, class_table, api_table):
    batch, hist = class_ids.shape
    n_rows = batch * hist
    assert n_rows % (NW * CHUNK) == 0
    n_chunks_per_w = n_rows // (NW * CHUNK)
    cls_flat = class_ids.reshape(n_rows // CHUNK, CHUNK).astype(jnp.int32)
    api_flat = api_ids.reshape(n_rows // CHUNK, CHUNK).astype(jnp.int32)
    out = _sc_embed(n_rows, n_chunks_per_w)(
        cls_flat, api_flat, class_table, api_table)
    return out.reshape(batch, hist, FINAL_DIM)


# speedup vs baseline: 3.5514x; 1.3441x over previous
import functools
import math

import jax
import jax.numpy as jnp
from jax import lax
from jax.experimental import pallas as pl
from jax.experimental.pallas import tpu as pltpu
from jax.experimental.pallas import tpu_sc as plsc

API_DIM = 32
CLASS_DIM = 32
FINAL_DIM = API_DIM + CLASS_DIM
SCALE = math.sqrt(FINAL_DIM)

NC = 2
NS = 16
NW = NC * NS
CHUNK = 128


def _sc_embed(n_rows, n_chunks_per_w):
    rows_per_w = n_chunks_per_w * CHUNK
    mesh = plsc.VectorSubcoreMesh(core_axis_name="c", subcore_axis_name="s")

    @functools.partial(
        pl.kernel,
        out_type=jax.ShapeDtypeStruct((n_rows, FINAL_DIM), jnp.float32),
        mesh=mesh,
        scratch_types=[
            pltpu.VMEM((n_chunks_per_w, CHUNK), jnp.int32),
            pltpu.VMEM((n_chunks_per_w, CHUNK), jnp.int32),
            pltpu.VMEM((CHUNK, CLASS_DIM), jnp.float32),
            pltpu.VMEM((CHUNK, API_DIM), jnp.float32),
            pltpu.VMEM((CHUNK, FINAL_DIM), jnp.float32),
            pltpu.VMEM((CHUNK, CLASS_DIM), jnp.float32),
            pltpu.VMEM((CHUNK, API_DIM), jnp.float32),
            pltpu.VMEM((CHUNK, FINAL_DIM), jnp.float32),
            pltpu.SemaphoreType.DMA,
            pltpu.SemaphoreType.DMA,
            pltpu.SemaphoreType.DMA,
            pltpu.SemaphoreType.DMA,
        ],
        compiler_params=pltpu.CompilerParams(use_tc_tiling_on_sc=False),
    )
    def k(cls_ids, api_ids, cls_tab, api_tab, out,
          idx_cls, idx_api, cls_v0, api_v0, out_v0, cls_v1, api_v1, out_v1,
          sem_g0, sem_g1, sem_w0, sem_w1):
        wid = lax.axis_index("s") * NC + lax.axis_index("c")
        cbase = wid * n_chunks_per_w
        rbase = wid * rows_per_w
        pltpu.sync_copy(cls_ids.at[pl.ds(cbase, n_chunks_per_w)], idx_cls)
        pltpu.sync_copy(api_ids.at[pl.ds(cbase, n_chunks_per_w)], idx_api)

        bufs = ((cls_v0, api_v0, out_v0, sem_g0, sem_w0),
                (cls_v1, api_v1, out_v1, sem_g1, sem_w1))

        def gather_start(j, p):
            cls_v, api_v, _, sem_g, _ = bufs[p]
            pltpu.async_copy(cls_tab.at[idx_cls.at[j]], cls_v, sem_g)
            pltpu.async_copy(api_tab.at[idx_api.at[j]], api_v, sem_g)

        gather_start(0, 0)

        def pair_body(jj, carry):
            for p in range(2):
                j = jj * 2 + p
                cls_v, api_v, out_v, sem_g, sem_w = bufs[p]

                @pl.when(j + 1 < n_chunks_per_w)
                def _():
                    gather_start(j + 1, 1 - p)

                pltpu.make_async_copy(
                    cls_tab.at[idx_cls.at[j]], cls_v, sem_g).wait()
                pltpu.make_async_copy(
                    api_tab.at[idx_api.at[j]], api_v, sem_g).wait()

                @pl.when(j >= 2)
                def _():
                    pltpu.make_async_copy(
                        out_v, out.at[pl.ds(rbase, CHUNK)], sem_w).wait()

                for i in range(CHUNK):
                    out_v[i, pl.ds(0, 16)] = cls_v[i, pl.ds(0, 16)] * SCALE
                    out_v[i, pl.ds(16, 16)] = cls_v[i, pl.ds(16, 16)] * SCALE
                    out_v[i, pl.ds(32, 16)] = api_v[i, pl.ds(0, 16)] * SCALE
                    out_v[i, pl.ds(48, 16)] = api_v[i, pl.ds(16, 16)] * SCALE

                pltpu.async_copy(
                    out_v, out.at[pl.ds(rbase + j * CHUNK, CHUNK)], sem_w)
            return carry

        lax.fori_loop(0, n_chunks_per_w // 2, pair_body, 0)
        for p in range(2):
            _, _, out_v, _, sem_w = bufs[p]
            pltpu.make_async_copy(
                out_v, out.at[pl.ds(rbase, CHUNK)], sem_w).wait()

    return k


def kernel(class_ids, api_ids, class_table, api_table):
    batch, hist = class_ids.shape
    n_rows = batch * hist
    assert n_rows % (NW * CHUNK) == 0
    n_chunks_per_w = n_rows // (NW * CHUNK)
    cls_flat = class_ids.reshape(n_rows // CHUNK, CHUNK).astype(jnp.int32)
    api_flat = api_ids.reshape(n_rows // CHUNK, CHUNK).astype(jnp.int32)
    out = _sc_embed(n_rows, n_chunks_per_w)(
        cls_flat, api_flat, class_table, api_table)
    return out.reshape(batch, hist, FINAL_DIM)
